# Initial kernel scaffold; baseline (speedup 1.0000x reference)
import jax
import jax.numpy as jnp
from jax.experimental import pallas as pl


def kernel(x, W1, g1, b1, W2, g2, b2):
    raise NotImplementedError("write your pallas kernel here")


# speedup vs baseline: 8.8357x; 8.8357x over previous
import functools

import jax
import jax.numpy as jnp
from jax import lax
from jax.experimental import pallas as pl
from jax.experimental.pallas import tpu as pltpu
from jax.experimental.pallas import tpu_sc as plsc

K = 16
TR = 256
TPT = 512

SC_NC = 2
SC_NS = 16
SC_NW = SC_NC * SC_NS


def _proj_topk_body(x_ref, xt_ref, wa_ref, wd_ref, idx_ref, q_ref, p_ref):
    b = pl.program_id(0)
    xb = x_ref[0]
    xt = xt_ref[0]
    q_ref[0] = jnp.dot(xt, wa_ref[...], preferred_element_type=jnp.float32)
    p_ref[0] = jnp.dot(xt, wd_ref[...], preferred_element_type=jnp.float32)
    n = xb.shape[1]
    sq_r = jnp.sum(xt * xt, axis=1, keepdims=True)
    sq_c = jnp.sum(xb * xb, axis=0, keepdims=True)
    d = sq_r - 2.0 * jnp.dot(xt, xb, preferred_element_type=jnp.float32) + sq_c
    cols = lax.broadcasted_iota(jnp.int32, (TR, n), 1)
    picks = []
    for t in range(K + 1):
        m = jnp.min(d, axis=1, keepdims=True)
        am = jnp.min(jnp.where(d == m, cols, n), axis=1, keepdims=True)
        if t > 0:
            picks.append(am)
        d = jnp.where(cols == am, jnp.inf, d)
    idx_ref[0] = jnp.concatenate(picks, axis=1) + b * n


def _proj_topk(x, xT, WaT, WdT):
    B, C, N = x.shape
    CP = WaT.shape[1]
    grid = (B, N // TR)
    return pl.pallas_call(
        _proj_topk_body,
        grid=grid,
        in_specs=[
            pl.BlockSpec((1, C, N), lambda b, t: (b, 0, 0)),
            pl.BlockSpec((1, TR, C), lambda b, t: (b, t, 0)),
            pl.BlockSpec((C, CP), lambda b, t: (0, 0)),
            pl.BlockSpec((C, CP), lambda b, t: (0, 0)),
        ],
        out_specs=[
            pl.BlockSpec((1, TR, K), lambda b, t: (b, t, 0)),
            pl.BlockSpec((1, TR, CP), lambda b, t: (b, t, 0)),
            pl.BlockSpec((1, TR, CP), lambda b, t: (b, t, 0)),
        ],
        out_shape=[
            jax.ShapeDtypeStruct((B, N, K), jnp.int32),
            jax.ShapeDtypeStruct((B, N, CP), jnp.float32),
            jax.ShapeDtypeStruct((B, N, CP), jnp.float32),
        ],
    )(x, xT, WaT, WdT)


def _sc_gather(table, idx):
    Q = idx.shape[0]
    C = table.shape[1]
    per_w = Q // SC_NW
    CH = 512
    nch = per_w // CH
    mesh = plsc.VectorSubcoreMesh(core_axis_name="c", subcore_axis_name="s")

    @functools.partial(
        pl.kernel,
        mesh=mesh,
        out_type=jax.ShapeDtypeStruct((Q, C), jnp.float32),
        scratch_types=[
            pltpu.VMEM((per_w,), jnp.int32),
            pltpu.VMEM((CH, C), jnp.float32),
            pltpu.SemaphoreType.DMA,
        ],
    )
    def gather_k(table_hbm, idx_hbm, out_hbm, idx_v, rows_v, sem):
        wid = lax.axis_index("s") * SC_NC + lax.axis_index("c")
        base = wid * per_w
        pltpu.sync_copy(idx_hbm.at[pl.ds(base, per_w)], idx_v)
        for c in range(nch):
            copies = []
            for j in range(CH // 128):
                off = c * CH + j * 128
                copies.append(pltpu.async_copy(
                    table_hbm.at[idx_v.at[pl.ds(off, 128)]],
                    rows_v.at[pl.ds(j * 128, 128)],
                    sem,
                ))
            for cp in copies:
                cp.wait()
            pltpu.sync_copy(rows_v, out_hbm.at[pl.ds(base + c * CH, CH)])

    return gather_k(table, idx)


def _stats1_body(g_ref, p_ref, s_ref, ss_ref):
    y = g_ref[...] + p_ref[...][None, :, :]
    s = jnp.sum(y, axis=(0, 1)).reshape(1, -1)
    ss = jnp.sum(y * y, axis=(0, 1)).reshape(1, -1)

    @pl.when(pl.program_id(0) == 0)
    def _():
        s_ref[...] = jnp.zeros_like(s_ref)
        ss_ref[...] = jnp.zeros_like(ss_ref)

    s_ref[...] += s
    ss_ref[...] += ss


def _stats1(g3, pT):
    BN, C = pT.shape
    grid = (BN // TPT,)
    return pl.pallas_call(
        _stats1_body,
        grid=grid,
        in_specs=[
            pl.BlockSpec((K, TPT, C), lambda i: (0, i, 0)),
            pl.BlockSpec((TPT, C), lambda i: (i, 0)),
        ],
        out_specs=[
            pl.BlockSpec((1, C), lambda i: (0, 0)),
            pl.BlockSpec((1, C), lambda i: (0, 0)),
        ],
        out_shape=[
            jax.ShapeDtypeStruct((1, C), jnp.float32),
            jax.ShapeDtypeStruct((1, C), jnp.float32),
        ],
    )(g3, pT)


def _main_body(nsamp, g_ref, p_ref, s_ref, ss_ref, g1_ref, b1_ref, w2t_ref,
               zmax_ref, zmin_ref, gram_ref, sa_ref):
    mean1 = s_ref[...] / nsamp
    var1 = ss_ref[...] / nsamp - mean1 * mean1
    scale1 = lax.rsqrt(var1 + 1e-5) * g1_ref[...]
    shift1 = b1_ref[...] - mean1 * scale1
    y = g_ref[...] + p_ref[...][None, :, :]
    a3 = y * scale1 + shift1
    a3 = jnp.where(a3 >= 0, a3, 0.2 * a3)
    a = a3.reshape(-1, a3.shape[-1])
    z3 = jnp.dot(a, w2t_ref[...], preferred_element_type=jnp.float32)
    z3 = z3.reshape(a3.shape)
    zmax_ref[...] = jnp.max(z3, axis=0)
    zmin_ref[...] = jnp.min(z3, axis=0)

    @pl.when(pl.program_id(0) == 0)
    def _():
        gram_ref[...] = jnp.zeros_like(gram_ref)
        sa_ref[...] = jnp.zeros_like(sa_ref)

    gram_ref[...] += lax.dot_general(
        a, a, (((0,), (0,)), ((), ())), preferred_element_type=jnp.float32)
    sa_ref[...] += jnp.sum(a, axis=0, keepdims=True)


def _main(g3, pT, s1, ss1, g1, b1, W2T, nsamp):
    BN, C = pT.shape
    grid = (BN // TPT,)
    return pl.pallas_call(
        functools.partial(_main_body, nsamp),
        grid=grid,
        in_specs=[
            pl.BlockSpec((K, TPT, C), lambda i: (0, i, 0)),
            pl.BlockSpec((TPT, C), lambda i: (i, 0)),
            pl.BlockSpec((1, C), lambda i: (0, 0)),
            pl.BlockSpec((1, C), lambda i: (0, 0)),
            pl.BlockSpec((1, C), lambda i: (0, 0)),
            pl.BlockSpec((1, C), lambda i: (0, 0)),
            pl.BlockSpec((C, C), lambda i: (0, 0)),
        ],
        out_specs=[
            pl.BlockSpec((TPT, C), lambda i: (i, 0)),
            pl.BlockSpec((TPT, C), lambda i: (i, 0)),
            pl.BlockSpec((C, C), lambda i: (0, 0)),
            pl.BlockSpec((1, C), lambda i: (0, 0)),
        ],
        out_shape=[
            jax.ShapeDtypeStruct((BN, C), jnp.float32),
            jax.ShapeDtypeStruct((BN, C), jnp.float32),
            jax.ShapeDtypeStruct((C, C), jnp.float32),
            jax.ShapeDtypeStruct((1, C), jnp.float32),
        ],
    )(g3, pT, s1, ss1, g1, b1, W2T)


def _final_body(nsamp, zmax_ref, zmin_ref, gram_ref, sa_ref, w2t_ref,
                g2_ref, b2_ref, out_ref):
    w2t = w2t_ref[...]
    mean_a = sa_ref[...] / nsamp
    mean_z = jnp.dot(mean_a, w2t, preferred_element_type=jnp.float32)
    r = jnp.dot(gram_ref[...], w2t, preferred_element_type=jnp.float32)
    ezz = jnp.sum(w2t * r, axis=0, keepdims=True) / nsamp
    var = ezz - mean_z * mean_z
    g2 = g2_ref[...]
    scale = lax.rsqrt(var + 1e-5) * g2
    shift = b2_ref[...] - mean_z * scale
    zext = jnp.where(g2 >= 0, zmax_ref[...], zmin_ref[...])
    tv = zext * scale + shift
    out_ref[...] = jnp.where(tv >= 0, tv, 0.2 * tv)


def _final(zmax, zmin, gram, sa, W2T, g2, b2, nsamp):
    BN, C = zmax.shape
    grid = (BN // TPT,)
    return pl.pallas_call(
        functools.partial(_final_body, nsamp),
        grid=grid,
        in_specs=[
            pl.BlockSpec((TPT, C), lambda i: (i, 0)),
            pl.BlockSpec((TPT, C), lambda i: (i, 0)),
            pl.BlockSpec((C, C), lambda i: (0, 0)),
            pl.BlockSpec((1, C), lambda i: (0, 0)),
            pl.BlockSpec((C, C), lambda i: (0, 0)),
            pl.BlockSpec((1, C), lambda i: (0, 0)),
            pl.BlockSpec((1, C), lambda i: (0, 0)),
        ],
        out_specs=pl.BlockSpec((TPT, C), lambda i: (i, 0)),
        out_shape=jax.ShapeDtypeStruct((BN, C), jnp.float32),
    )(zmax, zmin, gram, sa, W2T, g2, b2)


def kernel(x, W1, g1, b1, W2, g2, b2):
    B, C, N = x.shape
    CP = 128
    nsamp = float(B * N * K)
    xT = jnp.transpose(x, (0, 2, 1))
    pad_oc = [(0, 0), (0, CP - C)]
    WaT = jnp.pad(jnp.transpose(W1[:, :C]), pad_oc)
    WdT = jnp.pad(jnp.transpose(W1[:, C:] - W1[:, :C]), pad_oc)
    W2T = jnp.pad(jnp.transpose(W2), [(0, CP - C), (0, CP - C)])
    g1r = jnp.pad(g1, (0, CP - C)).reshape(1, CP)
    b1r = jnp.pad(b1, (0, CP - C)).reshape(1, CP)
    g2r = jnp.pad(g2, (0, CP - C)).reshape(1, CP)
    b2r = jnp.pad(b2, (0, CP - C)).reshape(1, CP)

    idxg, qT, pT = _proj_topk(x, xT, WaT, WdT)

    idx_flat = jnp.transpose(idxg, (2, 0, 1)).reshape(-1)
    g = _sc_gather(qT.reshape(B * N, CP), idx_flat)
    g3 = g.reshape(K, B * N, CP)
    pT2 = pT.reshape(B * N, CP)

    s1, ss1 = _stats1(g3, pT2)
    zmax, zmin, gram, sa = _main(g3, pT2, s1, ss1, g1r, b1r, W2T, nsamp)
    out = _final(zmax, zmin, gram, sa, W2T, g2r, b2r, nsamp)
    return jnp.transpose(out.reshape(B, N, CP), (0, 2, 1))[:, :C, :]


# speedup vs baseline: 10.4688x; 1.1848x over previous
import functools

import jax
import jax.numpy as jnp
from jax import lax
from jax.experimental import pallas as pl
from jax.experimental.pallas import tpu as pltpu
from jax.experimental.pallas import tpu_sc as plsc

K = 16
TR = 256
TPT = 512

SC_NC = 2
SC_NS = 16
SC_NW = SC_NC * SC_NS


def _proj_topk_body(x_ref, xt_ref, wa_ref, wd_ref, idx_ref, q_ref, p_ref):
    b = pl.program_id(0)
    xb = x_ref[0]
    xt = xt_ref[0]
    q_ref[0] = jnp.dot(xt, wa_ref[...], preferred_element_type=jnp.float32)
    p_ref[0] = jnp.dot(xt, wd_ref[...], preferred_element_type=jnp.float32)
    n = xb.shape[1]
    sq_r = jnp.sum(xt * xt, axis=1, keepdims=True)
    sq_c = jnp.sum(xb * xb, axis=0, keepdims=True)
    d = sq_r - 2.0 * jnp.dot(xt, xb, preferred_element_type=jnp.float32) + sq_c
    colsf = lax.broadcasted_iota(jnp.int32, (TR, n), 1).astype(jnp.float32)
    m = jnp.min(d, axis=1, keepdims=True)
    picks = []
    for t in range(K + 1):
        amf = jnp.min(jnp.where(d == m, colsf, float(n)), axis=1, keepdims=True)
        if t > 0:
            picks.append(amf)
        if t < K:
            d = jnp.where(colsf == amf, jnp.inf, d)
            m = jnp.min(d, axis=1, keepdims=True)
    idx_ref[0] = jnp.concatenate(picks, axis=1).astype(jnp.int32) + b * n


def _proj_topk(x, xT, WaT, WdT):
    B, C, N = x.shape
    CP = WaT.shape[1]
    grid = (B, N // TR)
    return pl.pallas_call(
        _proj_topk_body,
        grid=grid,
        in_specs=[
            pl.BlockSpec((1, C, N), lambda b, t: (b, 0, 0)),
            pl.BlockSpec((1, TR, C), lambda b, t: (b, t, 0)),
            pl.BlockSpec((C, CP), lambda b, t: (0, 0)),
            pl.BlockSpec((C, CP), lambda b, t: (0, 0)),
        ],
        out_specs=[
            pl.BlockSpec((1, TR, K), lambda b, t: (b, t, 0)),
            pl.BlockSpec((1, TR, CP), lambda b, t: (b, t, 0)),
            pl.BlockSpec((1, TR, CP), lambda b, t: (b, t, 0)),
        ],
        out_shape=[
            jax.ShapeDtypeStruct((B, N, K), jnp.int32),
            jax.ShapeDtypeStruct((B, N, CP), jnp.float32),
            jax.ShapeDtypeStruct((B, N, CP), jnp.float32),
        ],
    )(x, xT, WaT, WdT)


def _sc_gather(table, idx):
    Q = idx.shape[0]
    C = table.shape[1]
    per_w = Q // SC_NW
    CH = 512
    nch = per_w // CH
    mesh = plsc.VectorSubcoreMesh(core_axis_name="c", subcore_axis_name="s")

    @functools.partial(
        pl.kernel,
        mesh=mesh,
        out_type=jax.ShapeDtypeStruct((Q, C), jnp.float32),
        scratch_types=[
            pltpu.VMEM((per_w,), jnp.int32),
            pltpu.VMEM((CH, C), jnp.float32),
            pltpu.SemaphoreType.DMA,
        ],
    )
    def gather_k(table_hbm, idx_hbm, out_hbm, idx_v, rows_v, sem):
        wid = lax.axis_index("s") * SC_NC + lax.axis_index("c")
        base = wid * per_w
        pltpu.sync_copy(idx_hbm.at[pl.ds(base, per_w)], idx_v)
        for c in range(nch):
            copies = []
            for j in range(CH // 128):
                off = c * CH + j * 128
                copies.append(pltpu.async_copy(
                    table_hbm.at[idx_v.at[pl.ds(off, 128)]],
                    rows_v.at[pl.ds(j * 128, 128)],
                    sem,
                ))
            for cp in copies:
                cp.wait()
            pltpu.sync_copy(rows_v, out_hbm.at[pl.ds(base + c * CH, CH)])

    return gather_k(table, idx)


def _stats1_body(g_ref, p_ref, s_ref, ss_ref):
    y = g_ref[...] + p_ref[...][None, :, :]
    s = jnp.sum(y, axis=(0, 1)).reshape(1, -1)
    ss = jnp.sum(y * y, axis=(0, 1)).reshape(1, -1)

    @pl.when(pl.program_id(0) == 0)
    def _():
        s_ref[...] = jnp.zeros_like(s_ref)
        ss_ref[...] = jnp.zeros_like(ss_ref)

    s_ref[...] += s
    ss_ref[...] += ss


def _stats1(g3, pT):
    BN, C = pT.shape
    grid = (BN // TPT,)
    return pl.pallas_call(
        _stats1_body,
        grid=grid,
        in_specs=[
            pl.BlockSpec((K, TPT, C), lambda i: (0, i, 0)),
            pl.BlockSpec((TPT, C), lambda i: (i, 0)),
        ],
        out_specs=[
            pl.BlockSpec((1, C), lambda i: (0, 0)),
            pl.BlockSpec((1, C), lambda i: (0, 0)),
        ],
        out_shape=[
            jax.ShapeDtypeStruct((1, C), jnp.float32),
            jax.ShapeDtypeStruct((1, C), jnp.float32),
        ],
    )(g3, pT)


def _main_body(nsamp, g_ref, p_ref, s_ref, ss_ref, g1_ref, b1_ref, w2t_ref,
               zmax_ref, zmin_ref, gram_ref, sa_ref):
    mean1 = s_ref[...] / nsamp
    var1 = ss_ref[...] / nsamp - mean1 * mean1
    scale1 = lax.rsqrt(var1 + 1e-5) * g1_ref[...]
    shift1 = b1_ref[...] - mean1 * scale1
    y = g_ref[...] + p_ref[...][None, :, :]
    a3 = y * scale1 + shift1
    a3 = jnp.where(a3 >= 0, a3, 0.2 * a3)
    a = a3.reshape(-1, a3.shape[-1])
    z3 = jnp.dot(a, w2t_ref[...], preferred_element_type=jnp.float32)
    z3 = z3.reshape(a3.shape)
    zmax_ref[...] = jnp.max(z3, axis=0)
    zmin_ref[...] = jnp.min(z3, axis=0)

    @pl.when(pl.program_id(0) == 0)
    def _():
        gram_ref[...] = jnp.zeros_like(gram_ref)
        sa_ref[...] = jnp.zeros_like(sa_ref)

    gram_ref[...] += lax.dot_general(
        a, a, (((0,), (0,)), ((), ())), preferred_element_type=jnp.float32)
    sa_ref[...] += jnp.sum(a, axis=0, keepdims=True)


def _main(g3, pT, s1, ss1, g1, b1, W2T, nsamp):
    BN, C = pT.shape
    grid = (BN // TPT,)
    return pl.pallas_call(
        functools.partial(_main_body, nsamp),
        grid=grid,
        in_specs=[
            pl.BlockSpec((K, TPT, C), lambda i: (0, i, 0)),
            pl.BlockSpec((TPT, C), lambda i: (i, 0)),
            pl.BlockSpec((1, C), lambda i: (0, 0)),
            pl.BlockSpec((1, C), lambda i: (0, 0)),
            pl.BlockSpec((1, C), lambda i: (0, 0)),
            pl.BlockSpec((1, C), lambda i: (0, 0)),
            pl.BlockSpec((C, C), lambda i: (0, 0)),
        ],
        out_specs=[
            pl.BlockSpec((TPT, C), lambda i: (i, 0)),
            pl.BlockSpec((TPT, C), lambda i: (i, 0)),
            pl.BlockSpec((C, C), lambda i: (0, 0)),
            pl.BlockSpec((1, C), lambda i: (0, 0)),
        ],
        out_shape=[
            jax.ShapeDtypeStruct((BN, C), jnp.float32),
            jax.ShapeDtypeStruct((BN, C), jnp.float32),
            jax.ShapeDtypeStruct((C, C), jnp.float32),
            jax.ShapeDtypeStruct((1, C), jnp.float32),
        ],
    )(g3, pT, s1, ss1, g1, b1, W2T)


def _final_body(nsamp, zmax_ref, zmin_ref, gram_ref, sa_ref, w2t_ref,
                g2_ref, b2_ref, out_ref):
    w2t = w2t_ref[...]
    mean_a = sa_ref[...] / nsamp
    mean_z = jnp.dot(mean_a, w2t, preferred_element_type=jnp.float32)
    r = jnp.dot(gram_ref[...], w2t, preferred_element_type=jnp.float32)
    ezz = jnp.sum(w2t * r, axis=0, keepdims=True) / nsamp
    var = ezz - mean_z * mean_z
    g2 = g2_ref[...]
    scale = lax.rsqrt(var + 1e-5) * g2
    shift = b2_ref[...] - mean_z * scale
    zext = jnp.where(g2 >= 0, zmax_ref[...], zmin_ref[...])
    tv = zext * scale + shift
    out_ref[...] = jnp.where(tv >= 0, tv, 0.2 * tv)


def _final(zmax, zmin, gram, sa, W2T, g2, b2, nsamp):
    BN, C = zmax.shape
    grid = (BN // TPT,)
    return pl.pallas_call(
        functools.partial(_final_body, nsamp),
        grid=grid,
        in_specs=[
            pl.BlockSpec((TPT, C), lambda i: (i, 0)),
            pl.BlockSpec((TPT, C), lambda i: (i, 0)),
            pl.BlockSpec((C, C), lambda i: (0, 0)),
            pl.BlockSpec((1, C), lambda i: (0, 0)),
            pl.BlockSpec((C, C), lambda i: (0, 0)),
            pl.BlockSpec((1, C), lambda i: (0, 0)),
            pl.BlockSpec((1, C), lambda i: (0, 0)),
        ],
        out_specs=pl.BlockSpec((TPT, C), lambda i: (i, 0)),
        out_shape=jax.ShapeDtypeStruct((BN, C), jnp.float32),
    )(zmax, zmin, gram, sa, W2T, g2, b2)


def kernel(x, W1, g1, b1, W2, g2, b2):
    B, C, N = x.shape
    CP = 128
    nsamp = float(B * N * K)
    xT = jnp.transpose(x, (0, 2, 1))
    pad_oc = [(0, 0), (0, CP - C)]
    WaT = jnp.pad(jnp.transpose(W1[:, :C]), pad_oc)
    WdT = jnp.pad(jnp.transpose(W1[:, C:] - W1[:, :C]), pad_oc)
    W2T = jnp.pad(jnp.transpose(W2), [(0, CP - C), (0, CP - C)])
    g1r = jnp.pad(g1, (0, CP - C)).reshape(1, CP)
    b1r = jnp.pad(b1, (0, CP - C)).reshape(1, CP)
    g2r = jnp.pad(g2, (0, CP - C)).reshape(1, CP)
    b2r = jnp.pad(b2, (0, CP - C)).reshape(1, CP)

    idxg, qT, pT = _proj_topk(x, xT, WaT, WdT)

    idx_flat = jnp.transpose(idxg, (2, 0, 1)).reshape(-1)
    g = _sc_gather(qT.reshape(B * N, CP), idx_flat)
    g3 = g.reshape(K, B * N, CP)
    pT2 = pT.reshape(B * N, CP)

    s1, ss1 = _stats1(g3, pT2)
    zmax, zmin, gram, sa = _main(g3, pT2, s1, ss1, g1r, b1r, W2T, nsamp)
    out = _final(zmax, zmin, gram, sa, W2T, g2r, b2r, nsamp)
    return jnp.transpose(out.reshape(B, N, CP), (0, 2, 1))[:, :C, :]


# speedup vs baseline: 10.5631x; 1.0090x over previous
import functools

import jax
import jax.numpy as jnp
from jax import lax
from jax.experimental import pallas as pl
from jax.experimental.pallas import tpu as pltpu
from jax.experimental.pallas import tpu_sc as plsc

K = 16
TR = 256
TPT = 512

SC_NC = 2
SC_NS = 16
SC_NW = SC_NC * SC_NS


def _proj_topk_body(x_ref, xt_ref, wa_ref, wd_ref, idx_ref, q_ref, p_ref):
    b = pl.program_id(0)
    xb = x_ref[0]
    xt = xt_ref[0]
    q_ref[0] = jnp.dot(xt, wa_ref[...], preferred_element_type=jnp.float32)
    p_ref[0] = jnp.dot(xt, wd_ref[...], preferred_element_type=jnp.float32)
    n = xb.shape[1]
    sq_r = jnp.sum(xt * xt, axis=1, keepdims=True)
    sq_c = jnp.sum(xb * xb, axis=0, keepdims=True)
    d = sq_r - 2.0 * jnp.dot(xt, xb, preferred_element_type=jnp.float32) + sq_c
    colsf = lax.broadcasted_iota(jnp.int32, (TR, n), 1).astype(jnp.float32)
    m = jnp.min(d, axis=1, keepdims=True)
    picks = []
    for t in range(K + 1):
        amf = jnp.min(jnp.where(d == m, colsf, float(n)), axis=1, keepdims=True)
        if t > 0:
            picks.append(amf)
        if t < K:
            d = jnp.where(colsf == amf, jnp.inf, d)
            m = jnp.min(d, axis=1, keepdims=True)
    idx_ref[0] = jnp.concatenate(picks, axis=1).astype(jnp.int32) + b * n


def _proj_topk(x, xT, WaT, WdT):
    B, C, N = x.shape
    CP = WaT.shape[1]
    grid = (B, N // TR)
    return pl.pallas_call(
        _proj_topk_body,
        grid=grid,
        in_specs=[
            pl.BlockSpec((1, C, N), lambda b, t: (b, 0, 0)),
            pl.BlockSpec((1, TR, C), lambda b, t: (b, t, 0)),
            pl.BlockSpec((C, CP), lambda b, t: (0, 0)),
            pl.BlockSpec((C, CP), lambda b, t: (0, 0)),
        ],
        out_specs=[
            pl.BlockSpec((1, TR, K), lambda b, t: (b, t, 0)),
            pl.BlockSpec((1, TR, CP), lambda b, t: (b, t, 0)),
            pl.BlockSpec((1, TR, CP), lambda b, t: (b, t, 0)),
        ],
        out_shape=[
            jax.ShapeDtypeStruct((B, N, K), jnp.int32),
            jax.ShapeDtypeStruct((B, N, CP), jnp.float32),
            jax.ShapeDtypeStruct((B, N, CP), jnp.float32),
        ],
    )(x, xT, WaT, WdT)


def _sc_gather(table, idx):
    Q = idx.shape[0]
    C = table.shape[1]
    per_w = Q // SC_NW
    CH = 256
    nch = per_w // CH
    mesh = plsc.VectorSubcoreMesh(core_axis_name="c", subcore_axis_name="s")

    @functools.partial(
        pl.kernel,
        mesh=mesh,
        out_type=jax.ShapeDtypeStruct((Q, C), jnp.float32),
        scratch_types=[
            pltpu.VMEM((per_w,), jnp.int32),
            pltpu.VMEM((2, CH, C), jnp.float32),
            pltpu.SemaphoreType.DMA,
            pltpu.SemaphoreType.DMA,
            pltpu.SemaphoreType.DMA,
            pltpu.SemaphoreType.DMA,
        ],
    )
    def gather_k(table_hbm, idx_hbm, out_hbm, idx_v, rows_v, gs0, gs1, ws0, ws1):
        wid = lax.axis_index("s") * SC_NC + lax.axis_index("c")
        base = wid * per_w
        pltpu.sync_copy(idx_hbm.at[pl.ds(base, per_w)], idx_v)
        gsems = [gs0, gs1]
        wsems = [ws0, ws1]
        gh = [None, None]
        wh = [None, None]
        for c in range(nch + 1):
            if c < nch:
                p = c & 1
                if wh[p] is not None:
                    wh[p].wait()
                hs = []
                for j in range(CH // 128):
                    off = c * CH + j * 128
                    hs.append(pltpu.async_copy(
                        table_hbm.at[idx_v.at[pl.ds(off, 128)]],
                        rows_v.at[p, pl.ds(j * 128, 128)],
                        gsems[p],
                    ))
                gh[p] = hs
            if c >= 1:
                q = (c - 1) & 1
                for h in gh[q]:
                    h.wait()
                wh[q] = pltpu.async_copy(
                    rows_v.at[q],
                    out_hbm.at[pl.ds(base + (c - 1) * CH, CH)],
                    wsems[q],
                )
        for p in range(2):
            if wh[p] is not None:
                wh[p].wait()

    return gather_k(table, idx)


def _stats1_body(g_ref, p_ref, s_ref, ss_ref):
    y = g_ref[...] + p_ref[...][None, :, :]
    s = jnp.sum(y, axis=(0, 1)).reshape(1, -1)
    ss = jnp.sum(y * y, axis=(0, 1)).reshape(1, -1)

    @pl.when(pl.program_id(0) == 0)
    def _():
        s_ref[...] = jnp.zeros_like(s_ref)
        ss_ref[...] = jnp.zeros_like(ss_ref)

    s_ref[...] += s
    ss_ref[...] += ss


def _stats1(g3, pT):
    BN, C = pT.shape
    grid = (BN // TPT,)
    return pl.pallas_call(
        _stats1_body,
        grid=grid,
        in_specs=[
            pl.BlockSpec((K, TPT, C), lambda i: (0, i, 0)),
            pl.BlockSpec((TPT, C), lambda i: (i, 0)),
        ],
        out_specs=[
            pl.BlockSpec((1, C), lambda i: (0, 0)),
            pl.BlockSpec((1, C), lambda i: (0, 0)),
        ],
        out_shape=[
            jax.ShapeDtypeStruct((1, C), jnp.float32),
            jax.ShapeDtypeStruct((1, C), jnp.float32),
        ],
    )(g3, pT)


def _main_body(nsamp, g_ref, p_ref, s_ref, ss_ref, g1_ref, b1_ref, w2t_ref,
               zmax_ref, zmin_ref, gram_ref, sa_ref):
    mean1 = s_ref[...] / nsamp
    var1 = ss_ref[...] / nsamp - mean1 * mean1
    scale1 = lax.rsqrt(var1 + 1e-5) * g1_ref[...]
    shift1 = b1_ref[...] - mean1 * scale1
    y = g_ref[...] + p_ref[...][None, :, :]
    a3 = y * scale1 + shift1
    a3 = jnp.where(a3 >= 0, a3, 0.2 * a3)
    a = a3.reshape(-1, a3.shape[-1])
    z3 = jnp.dot(a, w2t_ref[...], preferred_element_type=jnp.float32)
    z3 = z3.reshape(a3.shape)
    zmax_ref[...] = jnp.max(z3, axis=0)
    zmin_ref[...] = jnp.min(z3, axis=0)

    @pl.when(pl.program_id(0) == 0)
    def _():
        gram_ref[...] = jnp.zeros_like(gram_ref)
        sa_ref[...] = jnp.zeros_like(sa_ref)

    gram_ref[...] += lax.dot_general(
        a, a, (((0,), (0,)), ((), ())), preferred_element_type=jnp.float32)
    sa_ref[...] += jnp.sum(a, axis=0, keepdims=True)


def _main(g3, pT, s1, ss1, g1, b1, W2T, nsamp):
    BN, C = pT.shape
    grid = (BN // TPT,)
    return pl.pallas_call(
        functools.partial(_main_body, nsamp),
        grid=grid,
        in_specs=[
            pl.BlockSpec((K, TPT, C), lambda i: (0, i, 0)),
            pl.BlockSpec((TPT, C), lambda i: (i, 0)),
            pl.BlockSpec((1, C), lambda i: (0, 0)),
            pl.BlockSpec((1, C), lambda i: (0, 0)),
            pl.BlockSpec((1, C), lambda i: (0, 0)),
            pl.BlockSpec((1, C), lambda i: (0, 0)),
            pl.BlockSpec((C, C), lambda i: (0, 0)),
        ],
        out_specs=[
            pl.BlockSpec((TPT, C), lambda i: (i, 0)),
            pl.BlockSpec((TPT, C), lambda i: (i, 0)),
            pl.BlockSpec((C, C), lambda i: (0, 0)),
            pl.BlockSpec((1, C), lambda i: (0, 0)),
        ],
        out_shape=[
            jax.ShapeDtypeStruct((BN, C), jnp.float32),
            jax.ShapeDtypeStruct((BN, C), jnp.float32),
            jax.ShapeDtypeStruct((C, C), jnp.float32),
            jax.ShapeDtypeStruct((1, C), jnp.float32),
        ],
    )(g3, pT, s1, ss1, g1, b1, W2T)


def _final_body(nsamp, zmax_ref, zmin_ref, gram_ref, sa_ref, w2t_ref,
                g2_ref, b2_ref, out_ref):
    w2t = w2t_ref[...]
    mean_a = sa_ref[...] / nsamp
    mean_z = jnp.dot(mean_a, w2t, preferred_element_type=jnp.float32)
    r = jnp.dot(gram_ref[...], w2t, preferred_element_type=jnp.float32)
    ezz = jnp.sum(w2t * r, axis=0, keepdims=True) / nsamp
    var = ezz - mean_z * mean_z
    g2 = g2_ref[...]
    scale = lax.rsqrt(var + 1e-5) * g2
    shift = b2_ref[...] - mean_z * scale
    zext = jnp.where(g2 >= 0, zmax_ref[...], zmin_ref[...])
    tv = zext * scale + shift
    out_ref[...] = jnp.where(tv >= 0, tv, 0.2 * tv)


def _final(zmax, zmin, gram, sa, W2T, g2, b2, nsamp):
    BN, C = zmax.shape
    grid = (BN // TPT,)
    return pl.pallas_call(
        functools.partial(_final_body, nsamp),
        grid=grid,
        in_specs=[
            pl.BlockSpec((TPT, C), lambda i: (i, 0)),
            pl.BlockSpec((TPT, C), lambda i: (i, 0)),
            pl.BlockSpec((C, C), lambda i: (0, 0)),
            pl.BlockSpec((1, C), lambda i: (0, 0)),
            pl.BlockSpec((C, C), lambda i: (0, 0)),
            pl.BlockSpec((1, C), lambda i: (0, 0)),
            pl.BlockSpec((1, C), lambda i: (0, 0)),
        ],
        out_specs=pl.BlockSpec((TPT, C), lambda i: (i, 0)),
        out_shape=jax.ShapeDtypeStruct((BN, C), jnp.float32),
    )(zmax, zmin, gram, sa, W2T, g2, b2)


def kernel(x, W1, g1, b1, W2, g2, b2):
    B, C, N = x.shape
    CP = 128
    nsamp = float(B * N * K)
    xT = jnp.transpose(x, (0, 2, 1))
    pad_oc = [(0, 0), (0, CP - C)]
    WaT = jnp.pad(jnp.transpose(W1[:, :C]), pad_oc)
    WdT = jnp.pad(jnp.transpose(W1[:, C:] - W1[:, :C]), pad_oc)
    W2T = jnp.pad(jnp.transpose(W2), [(0, CP - C), (0, CP - C)])
    g1r = jnp.pad(g1, (0, CP - C)).reshape(1, CP)
    b1r = jnp.pad(b1, (0, CP - C)).reshape(1, CP)
    g2r = jnp.pad(g2, (0, CP - C)).reshape(1, CP)
    b2r = jnp.pad(b2, (0, CP - C)).reshape(1, CP)

    idxg, qT, pT = _proj_topk(x, xT, WaT, WdT)

    idx_flat = jnp.transpose(idxg, (2, 0, 1)).reshape(-1)
    g = _sc_gather(qT.reshape(B * N, CP), idx_flat)
    g3 = g.reshape(K, B * N, CP)
    pT2 = pT.reshape(B * N, CP)

    s1, ss1 = _stats1(g3, pT2)
    zmax, zmin, gram, sa = _main(g3, pT2, s1, ss1, g1r, b1r, W2T, nsamp)
    out = _final(zmax, zmin, gram, sa, W2T, g2r, b2r, nsamp)
    return jnp.transpose(out.reshape(B, N, CP), (0, 2, 1))[:, :C, :]


# speedup vs baseline: 11.6451x; 1.1024x over previous
import functools

import jax
import jax.numpy as jnp
from jax import lax
from jax.experimental import pallas as pl
from jax.experimental.pallas import tpu as pltpu
from jax.experimental.pallas import tpu_sc as plsc

K = 16
TR = 256
TPT = 512

SC_NC = 2
SC_NS = 16
SC_NW = SC_NC * SC_NS


def _proj_topk_body(x_ref, xt_ref, wa_ref, wd_ref, idx_ref, q_ref, p_ref):
    b = pl.program_id(0)
    xb = x_ref[0]
    xt = xt_ref[0]
    q_ref[0] = jnp.dot(xt, wa_ref[...], preferred_element_type=jnp.float32)
    p_ref[0] = jnp.dot(xt, wd_ref[...], preferred_element_type=jnp.float32)
    n = xb.shape[1]
    sq_r = jnp.sum(xt * xt, axis=1, keepdims=True)
    sq_c = jnp.sum(xb * xb, axis=0, keepdims=True)
    d = sq_r - 2.0 * jnp.dot(xt, xb, preferred_element_type=jnp.float32) + sq_c
    colsf = lax.broadcasted_iota(jnp.int32, (TR, n), 1).astype(jnp.float32)
    m = jnp.min(d, axis=1, keepdims=True)
    picks = []
    for t in range(K + 1):
        amf = jnp.min(jnp.where(d == m, colsf, float(n)), axis=1, keepdims=True)
        if t > 0:
            picks.append(amf)
        if t < K:
            d = jnp.where(colsf == amf, jnp.inf, d)
            m = jnp.min(d, axis=1, keepdims=True)
    idx_ref[0] = (jnp.concatenate(picks, axis=1).astype(jnp.int32) + b * n) * 2


def _proj_topk(x, xT, WaT, WdT):
    B, C, N = x.shape
    CP = WaT.shape[1]
    grid = (B, N // TR)
    return pl.pallas_call(
        _proj_topk_body,
        grid=grid,
        in_specs=[
            pl.BlockSpec((1, C, N), lambda b, t: (b, 0, 0)),
            pl.BlockSpec((1, TR, C), lambda b, t: (b, t, 0)),
            pl.BlockSpec((C, CP), lambda b, t: (0, 0)),
            pl.BlockSpec((C, C), lambda b, t: (0, 0)),
        ],
        out_specs=[
            pl.BlockSpec((1, TR, K), lambda b, t: (b, t, 0)),
            pl.BlockSpec((1, TR, CP), lambda b, t: (b, t, 0)),
            pl.BlockSpec((1, TR, C), lambda b, t: (b, t, 0)),
        ],
        out_shape=[
            jax.ShapeDtypeStruct((B, N, K), jnp.int32),
            jax.ShapeDtypeStruct((B, N, CP), jnp.float32),
            jax.ShapeDtypeStruct((B, N, C), jnp.float32),
        ],
    )(x, xT, WaT, WdT)


def _sc_gather(table, idx):
    Q = idx.shape[0]
    C = table.shape[1]
    per_w = Q // SC_NW
    CH = 512
    nch = per_w // CH
    mesh = plsc.VectorSubcoreMesh(core_axis_name="c", subcore_axis_name="s")

    @functools.partial(
        pl.kernel,
        mesh=mesh,
        compiler_params=pltpu.CompilerParams(use_tc_tiling_on_sc=False),
        out_type=jax.ShapeDtypeStruct((Q, C), table.dtype),
        scratch_types=[
            pltpu.VMEM((per_w,), jnp.int32),
            pltpu.VMEM((2, CH, C), table.dtype),
            pltpu.SemaphoreType.DMA,
            pltpu.SemaphoreType.DMA,
            pltpu.SemaphoreType.DMA,
            pltpu.SemaphoreType.DMA,
        ],
    )
    def gather_k(table_hbm, idx_hbm, out_hbm, idx_v, rows_v, gs0, gs1, ws0, ws1):
        wid = lax.axis_index("s") * SC_NC + lax.axis_index("c")
        base = wid * per_w
        pltpu.sync_copy(idx_hbm.at[pl.ds(base, per_w)], idx_v)
        gsems = [gs0, gs1]
        wsems = [ws0, ws1]
        gh = [None, None]
        wh = [None, None]
        for c in range(nch + 1):
            if c < nch:
                p = c & 1
                if wh[p] is not None:
                    wh[p].wait()
                hs = []
                for j in range(CH // 128):
                    off = c * CH + j * 128
                    hs.append(pltpu.async_copy(
                        table_hbm.at[idx_v.at[pl.ds(off, 128)]],
                        rows_v.at[p, pl.ds(j * 128, 128)],
                        gsems[p],
                    ))
                gh[p] = hs
            if c >= 1:
                q = (c - 1) & 1
                for h in gh[q]:
                    h.wait()
                wh[q] = pltpu.async_copy(
                    rows_v.at[q],
                    out_hbm.at[pl.ds(base + (c - 1) * CH, CH)],
                    wsems[q],
                )
        for p in range(2):
            if wh[p] is not None:
                wh[p].wait()

    return gather_k(table, idx)


def _stats1_body(g_ref, p_ref, s_ref, ss_ref):
    c = p_ref.shape[-1]
    gr = g_ref[...]
    pb = p_ref[...][:, None, :]
    ya = gr[:, :, :c] + pb
    yb = gr[:, :, c:] + pb
    s = (jnp.sum(ya, axis=(0, 1)) + jnp.sum(yb, axis=(0, 1))).reshape(1, -1)
    ss = (jnp.sum(ya * ya, axis=(0, 1))
          + jnp.sum(yb * yb, axis=(0, 1))).reshape(1, -1)

    @pl.when(pl.program_id(0) == 0)
    def _():
        s_ref[...] = jnp.zeros_like(s_ref)
        ss_ref[...] = jnp.zeros_like(ss_ref)

    s_ref[...] += s
    ss_ref[...] += ss


def _stats1(g3, pT):
    BN, C = pT.shape
    grid = (BN // TPT,)
    return pl.pallas_call(
        _stats1_body,
        grid=grid,
        in_specs=[
            pl.BlockSpec((TPT, K // 2, 2 * C), lambda i: (i, 0, 0)),
            pl.BlockSpec((TPT, C), lambda i: (i, 0)),
        ],
        out_specs=[
            pl.BlockSpec((1, C), lambda i: (0, 0)),
            pl.BlockSpec((1, C), lambda i: (0, 0)),
        ],
        out_shape=[
            jax.ShapeDtypeStruct((1, C), jnp.float32),
            jax.ShapeDtypeStruct((1, C), jnp.float32),
        ],
    )(g3, pT)


def _main_body(nsamp, g_ref, p_ref, s_ref, ss_ref, g1_ref, b1_ref, w2t_ref,
               zmax_ref, zmin_ref, gram_ref, sa_ref):
    c = p_ref.shape[-1]
    mean1 = s_ref[...] / nsamp
    var1 = ss_ref[...] / nsamp - mean1 * mean1
    scale1 = lax.rsqrt(var1 + 1e-5) * g1_ref[...]
    shift1 = b1_ref[...] - mean1 * scale1
    gr = g_ref[...]
    pb = p_ref[...][:, None, :]
    w2t = w2t_ref[...]
    halves = []
    for ph in range(2):
        y = gr[:, :, ph * c:(ph + 1) * c] + pb
        a3 = y * scale1 + shift1
        a3 = jnp.where(a3 >= 0, a3, 0.2 * a3)
        halves.append(a3)
    aa = halves[0].reshape(-1, c)
    ab = halves[1].reshape(-1, c)
    za = jnp.dot(aa, w2t, preferred_element_type=jnp.float32)
    zb = jnp.dot(ab, w2t, preferred_element_type=jnp.float32)
    za3 = za.reshape(halves[0].shape)
    zb3 = zb.reshape(halves[0].shape)
    zmax_ref[...] = jnp.maximum(jnp.max(za3, axis=1), jnp.max(zb3, axis=1))
    zmin_ref[...] = jnp.minimum(jnp.min(za3, axis=1), jnp.min(zb3, axis=1))

    @pl.when(pl.program_id(0) == 0)
    def _():
        gram_ref[...] = jnp.zeros_like(gram_ref)
        sa_ref[...] = jnp.zeros_like(sa_ref)

    gram_ref[...] += (
        lax.dot_general(aa, aa, (((0,), (0,)), ((), ())),
                        preferred_element_type=jnp.float32)
        + lax.dot_general(ab, ab, (((0,), (0,)), ((), ())),
                          preferred_element_type=jnp.float32))
    sa_ref[...] += (jnp.sum(aa, axis=0, keepdims=True)
                    + jnp.sum(ab, axis=0, keepdims=True))


def _main(g3, pT, s1, ss1, g1, b1, W2T, nsamp):
    BN, C = pT.shape
    grid = (BN // TPT,)
    return pl.pallas_call(
        functools.partial(_main_body, nsamp),
        grid=grid,
        in_specs=[
            pl.BlockSpec((TPT, K // 2, 2 * C), lambda i: (i, 0, 0)),
            pl.BlockSpec((TPT, C), lambda i: (i, 0)),
            pl.BlockSpec((1, C), lambda i: (0, 0)),
            pl.BlockSpec((1, C), lambda i: (0, 0)),
            pl.BlockSpec((1, C), lambda i: (0, 0)),
            pl.BlockSpec((1, C), lambda i: (0, 0)),
            pl.BlockSpec((C, C), lambda i: (0, 0)),
        ],
        out_specs=[
            pl.BlockSpec((TPT, C), lambda i: (i, 0)),
            pl.BlockSpec((TPT, C), lambda i: (i, 0)),
            pl.BlockSpec((C, C), lambda i: (0, 0)),
            pl.BlockSpec((1, C), lambda i: (0, 0)),
        ],
        out_shape=[
            jax.ShapeDtypeStruct((BN, C), jnp.float32),
            jax.ShapeDtypeStruct((BN, C), jnp.float32),
            jax.ShapeDtypeStruct((C, C), jnp.float32),
            jax.ShapeDtypeStruct((1, C), jnp.float32),
        ],
    )(g3, pT, s1, ss1, g1, b1, W2T)


def _final_body(nsamp, zmax_ref, zmin_ref, gram_ref, sa_ref, w2t_ref,
                g2_ref, b2_ref, out_ref):
    w2t = w2t_ref[...]
    mean_a = sa_ref[...] / nsamp
    mean_z = jnp.dot(mean_a, w2t, preferred_element_type=jnp.float32)
    r = jnp.dot(gram_ref[...], w2t, preferred_element_type=jnp.float32)
    ezz = jnp.sum(w2t * r, axis=0, keepdims=True) / nsamp
    var = ezz - mean_z * mean_z
    g2 = g2_ref[...]
    scale = lax.rsqrt(var + 1e-5) * g2
    shift = b2_ref[...] - mean_z * scale
    zext = jnp.where(g2 >= 0, zmax_ref[...], zmin_ref[...])
    tv = zext * scale + shift
    out_ref[...] = jnp.where(tv >= 0, tv, 0.2 * tv)


def _final(zmax, zmin, gram, sa, W2T, g2, b2, nsamp):
    BN, C = zmax.shape
    grid = (BN // TPT,)
    return pl.pallas_call(
        functools.partial(_final_body, nsamp),
        grid=grid,
        in_specs=[
            pl.BlockSpec((TPT, C), lambda i: (i, 0)),
            pl.BlockSpec((TPT, C), lambda i: (i, 0)),
            pl.BlockSpec((C, C), lambda i: (0, 0)),
            pl.BlockSpec((1, C), lambda i: (0, 0)),
            pl.BlockSpec((C, C), lambda i: (0, 0)),
            pl.BlockSpec((1, C), lambda i: (0, 0)),
            pl.BlockSpec((1, C), lambda i: (0, 0)),
        ],
        out_specs=pl.BlockSpec((TPT, C), lambda i: (i, 0)),
        out_shape=jax.ShapeDtypeStruct((BN, C), jnp.float32),
    )(zmax, zmin, gram, sa, W2T, g2, b2)


def kernel(x, W1, g1, b1, W2, g2, b2):
    B, C, N = x.shape
    CP = 128
    nsamp = float(B * N * K)
    xT = jnp.transpose(x, (0, 2, 1))
    WaT = jnp.pad(jnp.transpose(W1[:, :C]), [(0, 0), (0, CP - C)])
    WdT = jnp.transpose(W1[:, C:] - W1[:, :C])
    W2T = jnp.transpose(W2)
    g1r = g1.reshape(1, C)
    b1r = b1.reshape(1, C)
    g2r = g2.reshape(1, C)
    b2r = b2.reshape(1, C)

    idxg, qT, pT = _proj_topk(x, xT, WaT, WdT)

    idx_flat = idxg.reshape(-1)
    g = _sc_gather(qT.reshape(2 * B * N, C), idx_flat)
    g3 = g.reshape(B * N, K // 2, 2 * C)
    pT2 = pT.reshape(B * N, C)

    s1, ss1 = _stats1(g3, pT2)
    zmax, zmin, gram, sa = _main(g3, pT2, s1, ss1, g1r, b1r, W2T, nsamp)
    out = _final(zmax, zmin, gram, sa, W2T, g2r, b2r, nsamp)
    return jnp.transpose(out.reshape(B, N, C), (0, 2, 1))


# speedup vs baseline: 12.8462x; 1.1031x over previous
import functools

import jax
import jax.numpy as jnp
from jax import lax
from jax.experimental import pallas as pl
from jax.experimental.pallas import tpu as pltpu
from jax.experimental.pallas import tpu_sc as plsc

K = 16
TR = 256
TPT = 512

SC_NC = 2
SC_NS = 16
SC_NW = SC_NC * SC_NS


def _proj_topk_body(x_ref, xt_ref, wa_ref, wd_ref, idx_ref, q_ref, p_ref):
    b = pl.program_id(0)
    xb = x_ref[0]
    xt = xt_ref[0]
    q_ref[0] = jnp.dot(xt, wa_ref[...], preferred_element_type=jnp.float32)
    p_ref[0] = jnp.dot(xt, wd_ref[...], preferred_element_type=jnp.float32)
    n = xb.shape[1]
    sq_r = jnp.sum(xt * xt, axis=1, keepdims=True)
    sq_c = jnp.sum(xb * xb, axis=0, keepdims=True)
    d = sq_r - 2.0 * jnp.dot(xt, xb, preferred_element_type=jnp.float32) + sq_c
    colsf = lax.broadcasted_iota(jnp.int32, (TR, n), 1).astype(jnp.float32)
    m = jnp.min(d, axis=1, keepdims=True)
    picks = []
    for t in range(K + 1):
        amf = jnp.min(jnp.where(d == m, colsf, float(n)), axis=1, keepdims=True)
        if t > 0:
            picks.append(amf)
        if t < K:
            d = jnp.where(colsf == amf, jnp.inf, d)
            m = jnp.min(d, axis=1, keepdims=True)
    idx_ref[0] = (jnp.concatenate(picks, axis=1).astype(jnp.int32) + b * n) * 2


def _proj_topk(x, xT, WaT, WdT):
    B, C, N = x.shape
    CP = WaT.shape[1]
    grid = (B, N // TR)
    return pl.pallas_call(
        _proj_topk_body,
        grid=grid,
        in_specs=[
            pl.BlockSpec((1, C, N), lambda b, t: (b, 0, 0)),
            pl.BlockSpec((1, TR, C), lambda b, t: (b, t, 0)),
            pl.BlockSpec((C, CP), lambda b, t: (0, 0)),
            pl.BlockSpec((C, C), lambda b, t: (0, 0)),
        ],
        out_specs=[
            pl.BlockSpec((1, TR, K), lambda b, t: (b, t, 0)),
            pl.BlockSpec((1, TR, CP), lambda b, t: (b, t, 0)),
            pl.BlockSpec((1, TR, C), lambda b, t: (b, t, 0)),
        ],
        out_shape=[
            jax.ShapeDtypeStruct((B, N, K), jnp.int32),
            jax.ShapeDtypeStruct((B, N, CP), jnp.float32),
            jax.ShapeDtypeStruct((B, N, C), jnp.float32),
        ],
    )(x, xT, WaT, WdT)


def _sc_gather(table, idx):
    Q = idx.shape[0]
    C = table.shape[1]
    per_w = Q // SC_NW
    CH = 512
    nch = per_w // CH
    mesh = plsc.VectorSubcoreMesh(core_axis_name="c", subcore_axis_name="s")

    @functools.partial(
        pl.kernel,
        mesh=mesh,
        compiler_params=pltpu.CompilerParams(use_tc_tiling_on_sc=False),
        out_type=jax.ShapeDtypeStruct((Q, C), table.dtype),
        scratch_types=[
            pltpu.VMEM((per_w,), jnp.int32),
            pltpu.VMEM((2, CH, C), table.dtype),
            pltpu.SemaphoreType.DMA,
            pltpu.SemaphoreType.DMA,
            pltpu.SemaphoreType.DMA,
            pltpu.SemaphoreType.DMA,
        ],
    )
    def gather_k(table_hbm, idx_hbm, out_hbm, idx_v, rows_v, gs0, gs1, ws0, ws1):
        wid = lax.axis_index("s") * SC_NC + lax.axis_index("c")
        base = wid * per_w
        pltpu.sync_copy(idx_hbm.at[pl.ds(base, per_w)], idx_v)
        gsems = [gs0, gs1]
        wsems = [ws0, ws1]
        gh = [None, None]
        wh = [None, None]
        for c in range(nch + 1):
            if c < nch:
                p = c & 1
                if wh[p] is not None:
                    wh[p].wait()
                hs = []
                for j in range(CH // 128):
                    off = c * CH + j * 128
                    hs.append(pltpu.async_copy(
                        table_hbm.at[idx_v.at[pl.ds(off, 128)]],
                        rows_v.at[p, pl.ds(j * 128, 128)],
                        gsems[p],
                    ))
                gh[p] = hs
            if c >= 1:
                q = (c - 1) & 1
                for h in gh[q]:
                    h.wait()
                wh[q] = pltpu.async_copy(
                    rows_v.at[q],
                    out_hbm.at[pl.ds(base + (c - 1) * CH, CH)],
                    wsems[q],
                )
        for p in range(2):
            if wh[p] is not None:
                wh[p].wait()

    return gather_k(table, idx)


def _stats1_body(g_ref, p_ref, s_ref, ss_ref):
    y = g_ref[...] + p_ref[...][None, :, :]
    s = jnp.sum(y, axis=(0, 1)).reshape(1, -1)
    ss = jnp.sum(y * y, axis=(0, 1)).reshape(1, -1)

    @pl.when(pl.program_id(0) == 0)
    def _():
        s_ref[...] = jnp.zeros_like(s_ref)
        ss_ref[...] = jnp.zeros_like(ss_ref)

    s_ref[...] += s
    ss_ref[...] += ss


def _stats1(g3, pp):
    BN2, C2 = pp.shape
    tpt2 = TPT // 2
    grid = (BN2 // tpt2,)
    return pl.pallas_call(
        _stats1_body,
        grid=grid,
        in_specs=[
            pl.BlockSpec((K, tpt2, C2), lambda i: (0, i, 0)),
            pl.BlockSpec((tpt2, C2), lambda i: (i, 0)),
        ],
        out_specs=[
            pl.BlockSpec((1, C2), lambda i: (0, 0)),
            pl.BlockSpec((1, C2), lambda i: (0, 0)),
        ],
        out_shape=[
            jax.ShapeDtypeStruct((1, C2), jnp.float32),
            jax.ShapeDtypeStruct((1, C2), jnp.float32),
        ],
    )(g3, pp)


def _main_body(nsamp, g_ref, p_ref, s_ref, ss_ref, g1_ref, b1_ref, w2d_ref,
               zmax_ref, zmin_ref, gram_ref, sa_ref):
    c2 = g_ref.shape[-1]
    c = c2 // 2
    sf = s_ref[...][:, :c] + s_ref[...][:, c:]
    ssf = ss_ref[...][:, :c] + ss_ref[...][:, c:]
    mean1 = sf / nsamp
    var1 = ssf / nsamp - mean1 * mean1
    sc64 = lax.rsqrt(var1 + 1e-5) * g1_ref[...]
    sh64 = b1_ref[...] - mean1 * sc64
    scale1 = jnp.concatenate([sc64, sc64], axis=1)
    shift1 = jnp.concatenate([sh64, sh64], axis=1)
    y = g_ref[...] + p_ref[...][None, :, :]
    a3 = y * scale1 + shift1
    a3 = jnp.where(a3 >= 0, a3, 0.2 * a3)
    a = a3.reshape(-1, c2)
    z3 = jnp.dot(a, w2d_ref[...], preferred_element_type=jnp.float32)
    z3 = z3.reshape(a3.shape)
    zmax_ref[...] = jnp.max(z3, axis=0)
    zmin_ref[...] = jnp.min(z3, axis=0)

    @pl.when(pl.program_id(0) == 0)
    def _():
        gram_ref[...] = jnp.zeros_like(gram_ref)
        sa_ref[...] = jnp.zeros_like(sa_ref)

    gram_ref[...] += lax.dot_general(
        a, a, (((0,), (0,)), ((), ())), preferred_element_type=jnp.float32)
    sa_ref[...] += jnp.sum(a, axis=0, keepdims=True)


def _main(g3, pp, s1, ss1, g1, b1, W2D, nsamp):
    BN2, C2 = pp.shape
    C = C2 // 2
    tpt2 = TPT // 2
    grid = (BN2 // tpt2,)
    return pl.pallas_call(
        functools.partial(_main_body, nsamp),
        grid=grid,
        in_specs=[
            pl.BlockSpec((K, tpt2, C2), lambda i: (0, i, 0)),
            pl.BlockSpec((tpt2, C2), lambda i: (i, 0)),
            pl.BlockSpec((1, C2), lambda i: (0, 0)),
            pl.BlockSpec((1, C2), lambda i: (0, 0)),
            pl.BlockSpec((1, C), lambda i: (0, 0)),
            pl.BlockSpec((1, C), lambda i: (0, 0)),
            pl.BlockSpec((C2, C2), lambda i: (0, 0)),
        ],
        out_specs=[
            pl.BlockSpec((tpt2, C2), lambda i: (i, 0)),
            pl.BlockSpec((tpt2, C2), lambda i: (i, 0)),
            pl.BlockSpec((C2, C2), lambda i: (0, 0)),
            pl.BlockSpec((1, C2), lambda i: (0, 0)),
        ],
        out_shape=[
            jax.ShapeDtypeStruct((BN2, C2), jnp.float32),
            jax.ShapeDtypeStruct((BN2, C2), jnp.float32),
            jax.ShapeDtypeStruct((C2, C2), jnp.float32),
            jax.ShapeDtypeStruct((1, C2), jnp.float32),
        ],
    )(g3, pp, s1, ss1, g1, b1, W2D)


def _final_body(nsamp, zmax_ref, zmin_ref, gram_ref, sa_ref, w2t_ref,
                g2_ref, b2_ref, out_ref):
    c2 = zmax_ref.shape[-1]
    c = c2 // 2
    w2t = w2t_ref[...]
    gram = gram_ref[...]
    g64 = gram[:c, :c] + gram[c:, c:]
    mean_a = (sa_ref[...][:, :c] + sa_ref[...][:, c:]) / nsamp
    mean_z = jnp.dot(mean_a, w2t, preferred_element_type=jnp.float32)
    r = jnp.dot(g64, w2t, preferred_element_type=jnp.float32)
    ezz = jnp.sum(w2t * r, axis=0, keepdims=True) / nsamp
    var = ezz - mean_z * mean_z
    g2 = g2_ref[...]
    sc64 = lax.rsqrt(var + 1e-5) * g2
    sh64 = b2_ref[...] - mean_z * sc64
    scale = jnp.concatenate([sc64, sc64], axis=1)
    shift = jnp.concatenate([sh64, sh64], axis=1)
    g2d = jnp.concatenate([g2, g2], axis=1)
    zext = jnp.where(g2d >= 0, zmax_ref[...], zmin_ref[...])
    tv = zext * scale + shift
    out_ref[...] = jnp.where(tv >= 0, tv, 0.2 * tv)


def _final(zmax, zmin, gram, sa, W2T, g2, b2, nsamp):
    BN2, C2 = zmax.shape
    C = C2 // 2
    tpt2 = TPT // 2
    grid = (BN2 // tpt2,)
    return pl.pallas_call(
        functools.partial(_final_body, nsamp),
        grid=grid,
        in_specs=[
            pl.BlockSpec((tpt2, C2), lambda i: (i, 0)),
            pl.BlockSpec((tpt2, C2), lambda i: (i, 0)),
            pl.BlockSpec((C2, C2), lambda i: (0, 0)),
            pl.BlockSpec((1, C2), lambda i: (0, 0)),
            pl.BlockSpec((C, C), lambda i: (0, 0)),
            pl.BlockSpec((1, C), lambda i: (0, 0)),
            pl.BlockSpec((1, C), lambda i: (0, 0)),
        ],
        out_specs=pl.BlockSpec((tpt2, C2), lambda i: (i, 0)),
        out_shape=jax.ShapeDtypeStruct((BN2, C2), jnp.float32),
    )(zmax, zmin, gram, sa, W2T, g2, b2)


def kernel(x, W1, g1, b1, W2, g2, b2):
    B, C, N = x.shape
    CP = 128
    nsamp = float(B * N * K)
    xT = jnp.transpose(x, (0, 2, 1))
    WaT = jnp.pad(jnp.transpose(W1[:, :C]), [(0, 0), (0, CP - C)])
    WdT = jnp.transpose(W1[:, C:] - W1[:, :C])
    W2T = jnp.transpose(W2)
    W2D = jnp.zeros((CP, CP), jnp.float32)
    W2D = W2D.at[:C, :C].set(W2T).at[C:, C:].set(W2T)
    g1r = g1.reshape(1, C)
    b1r = b1.reshape(1, C)
    g2r = g2.reshape(1, C)
    b2r = b2.reshape(1, C)

    idxg, qT, pT = _proj_topk(x, xT, WaT, WdT)

    idx_flat = jnp.transpose(idxg, (2, 0, 1)).reshape(-1)
    g = _sc_gather(qT.reshape(2 * B * N, C), idx_flat)
    g3 = g.reshape(K, B * N // 2, 2 * C)
    pp = pT.reshape(B * N // 2, 2 * C)

    s1, ss1 = _stats1(g3, pp)
    zmax, zmin, gram, sa = _main(g3, pp, s1, ss1, g1r, b1r, W2D, nsamp)
    out = _final(zmax, zmin, gram, sa, W2T, g2r, b2r, nsamp)
    return jnp.transpose(out.reshape(B, N, C), (0, 2, 1))
